# baseline (speedup 1.0000x reference)
import functools

import jax
import jax.numpy as jnp
from jax import lax
from jax.experimental import pallas as pl
from jax.experimental.pallas import tpu as pltpu
from jax.experimental.pallas import tpu_sc as plsc

_N = 10000
_E = 160000
_D = 128

_NC = 2
_NS = 16
_RP = 624
_RTAIL = _N - _RP * _NS
_CH = 128
_NCHUNK = _E // _CH
_QB = 2
_NQ = _NCHUNK // _QB
_DEGW = 16

_HALF = _N // _NC
_TRASH = _HALF
_ACCR = _HALF + 8
_RPH = 312
_RHTAIL = _HALF - _RPH * _NS

_sc_mesh = functools.partial(
    plsc.VectorSubcoreMesh, core_axis_name="c", subcore_axis_name="s")


def _deg_counts(dst2):
    @functools.partial(
        pl.kernel,
        mesh=_sc_mesh(),
        out_type=jax.ShapeDtypeStruct((_N, _DEGW), jnp.float32),
        scratch_types=[
            pltpu.VMEM((_QB, _CH), jnp.int32),
            pltpu.VMEM((_CH, _DEGW), jnp.float32),
            pltpu.VMEM((_RPH, _DEGW), jnp.float32),
            pltpu.VMEM_SHARED((_ACCR, _DEGW), jnp.float32),
            pltpu.SemaphoreType.DMA,
        ],
    )
    def k(dst_hbm, out_hbm, didx, ones_v, zbuf, acc, sem):
        c = lax.axis_index("c")
        s = lax.axis_index("s")

        def fill(i, carry):
            @pl.when(i < _CH)
            def _():
                ones_v[i] = jnp.ones((_DEGW,), jnp.float32)
            zbuf[i] = jnp.zeros((_DEGW,), jnp.float32)
            return carry

        lax.fori_loop(0, _RPH, fill, 0)
        pltpu.sync_copy(zbuf, acc.at[pl.ds(s * _RPH, _RPH)])

        @pl.when(s == 0)
        def _():
            pltpu.sync_copy(zbuf.at[pl.ds(0, _ACCR - _RPH * _NS)],
                            acc.at[pl.ds(_RPH * _NS, _ACCR - _RPH * _NS)])

        plsc.subcore_barrier()

        def body(j, carry):
            q = s + _NS * j

            @pl.when(q < _NQ)
            def _():
                pltpu.sync_copy(dst_hbm.at[q], didx)
                for r in range(_QB):
                    _localize(didx, c, r)
                    pltpu.sync_copy(ones_v, acc.at[didx.at[r]], add=True)

            return carry

        lax.fori_loop(0, (_NQ + _NS - 1) // _NS, body, 0)
        plsc.subcore_barrier()
        pltpu.sync_copy(acc.at[pl.ds(s * _RPH, _RPH)], zbuf)
        pltpu.sync_copy(zbuf, out_hbm.at[pl.ds(c * _HALF + s * _RPH, _RPH)])

        @pl.when(s == 0)
        def _():
            pltpu.sync_copy(acc.at[pl.ds(_RPH * _NS, _RHTAIL)],
                            zbuf.at[pl.ds(0, _RHTAIL)])
            pltpu.sync_copy(zbuf.at[pl.ds(0, _RHTAIL)],
                            out_hbm.at[pl.ds(c * _HALF + _RPH * _NS, _RHTAIL)])

    return k(dst2)


def _localize(didx, c, r=0):
    lo = c * _HALF
    for kk in range(_CH // 16):
        v = didx[r, pl.ds(kk * 16, 16)]
        vl = v - lo
        inb = jnp.logical_and(vl >= 0, vl < _HALF)
        didx[r, pl.ds(kk * 16, 16)] = jnp.where(inb, vl, _TRASH)


def _msg_sums(src2, dst2, g):
    @functools.partial(
        pl.kernel,
        mesh=_sc_mesh(),
        out_type=jax.ShapeDtypeStruct((_N, _D), jnp.float32),
        scratch_types=[
            pltpu.VMEM((_CH,), jnp.int32),
            pltpu.VMEM((1, _CH), jnp.int32),
            pltpu.VMEM((_CH, _D), jnp.float32),
            pltpu.VMEM((_RPH, _D), jnp.float32),
            pltpu.VMEM_SHARED((_ACCR, _D), jnp.float32),
            pltpu.SemaphoreType.DMA,
        ],
    )
    def k(src_hbm, dst_hbm, g_hbm, out_hbm, sidx, didx, rows, zbuf, acc, sem):
        c = lax.axis_index("c")
        s = lax.axis_index("s")

        def fill(i, carry):
            for jj in range(_D // 16):
                zbuf[i, pl.ds(jj * 16, 16)] = jnp.zeros((16,), jnp.float32)
            return carry

        lax.fori_loop(0, _RPH, fill, 0)
        pltpu.sync_copy(zbuf, acc.at[pl.ds(s * _RPH, _RPH)])

        @pl.when(s == 0)
        def _():
            pltpu.sync_copy(zbuf.at[pl.ds(0, _ACCR - _RPH * _NS)],
                            acc.at[pl.ds(_RPH * _NS, _ACCR - _RPH * _NS)])

        plsc.subcore_barrier()

        def body(j, carry):
            ch = s + _NS * j

            @pl.when(ch < _NCHUNK)
            def _():
                base = ch * _CH
                pltpu.sync_copy(src_hbm.at[pl.ds(base, _CH)], sidx)
                pltpu.sync_copy(dst_hbm.at[pl.ds(base, _CH)], didx.at[0])
                _localize(didx, c)
                pltpu.async_copy(g_hbm.at[sidx], rows, sem).wait()
                pltpu.sync_copy(rows, acc.at[didx.at[0]], add=True)

            return carry

        lax.fori_loop(0, (_NCHUNK + _NS - 1) // _NS, body, 0)
        plsc.subcore_barrier()
        pltpu.sync_copy(acc.at[pl.ds(s * _RPH, _RPH)], zbuf)
        pltpu.sync_copy(zbuf, out_hbm.at[pl.ds(c * _HALF + s * _RPH, _RPH)])

        @pl.when(s == 0)
        def _():
            pltpu.sync_copy(acc.at[pl.ds(_RPH * _NS, _RHTAIL)],
                            zbuf.at[pl.ds(0, _RHTAIL)])
            pltpu.sync_copy(zbuf.at[pl.ds(0, _RHTAIL)],
                            out_hbm.at[pl.ds(c * _HALF + _RPH * _NS, _RHTAIL)])

    return k(src2, dst2, g)


_BM = 2000
_BMM = 400


def _prep_kernel(x_ref, wc_ref, wh_ref, degc_ref, g_ref, h_ref):
    deg = degc_ref[:, 0] + 1.0
    dis = lax.rsqrt(deg)
    xb = x_ref[...]
    hc = lax.dot_general(xb, wc_ref[...], (((1,), (1,)), ((), ())),
                         preferred_element_type=jnp.float32)
    g_ref[...] = hc * dis[:, None]
    hh = lax.dot_general(xb, wh_ref[...], (((1,), (1,)), ((), ())),
                         preferred_element_type=jnp.float32)
    h_ref[...] = jnp.maximum(hh, 0.0)


def _prep(x, W_conv, W_high, degp):
    grid = (_N // _BM,)
    return pl.pallas_call(
        _prep_kernel,
        grid=grid,
        in_specs=[
            pl.BlockSpec((_BM, _D), lambda i: (i, 0)),
            pl.BlockSpec((_D, _D), lambda i: (0, 0)),
            pl.BlockSpec((_D, _D), lambda i: (0, 0)),
            pl.BlockSpec((_BM, _DEGW), lambda i: (i, 0)),
        ],
        out_specs=[
            pl.BlockSpec((_BM, _D), lambda i: (i, 0)),
            pl.BlockSpec((_BM, _D), lambda i: (i, 0)),
        ],
        out_shape=[
            jax.ShapeDtypeStruct((_N, _D), jnp.float32),
            jax.ShapeDtypeStruct((_N, _D), jnp.float32),
        ],
    )(x, W_conv, W_high, degp)


def _mm_kernel(a_ref, b_ref, o_ref):
    o_ref[...] = jnp.dot(a_ref[...], b_ref[...],
                         preferred_element_type=jnp.float32)


def _mm(a, b):
    grid = (_N // _BMM,)
    return pl.pallas_call(
        _mm_kernel,
        grid=grid,
        in_specs=[
            pl.BlockSpec((_BMM, _N), lambda i: (i, 0)),
            pl.BlockSpec((_N, _D), lambda i: (0, 0)),
        ],
        out_specs=pl.BlockSpec((_BMM, _D), lambda i: (i, 0)),
        out_shape=jax.ShapeDtypeStruct((_N, _D), jnp.float32),
        compiler_params=pltpu.CompilerParams(
            dimension_semantics=("arbitrary",)),
    )(a, b)


def _combine_kernel(t3_ref, sp_ref, g_ref, degc_ref, b_ref, sc_ref, o_ref):
    deg = degc_ref[:, 0] + 1.0
    dis = lax.rsqrt(deg)
    S = sp_ref[...] + g_ref[...]
    gcn = S * dis[:, None] + b_ref[...]
    hl = jnp.maximum(gcn, 0.0)
    o_ref[...] = sc_ref[0, 0] * hl + sc_ref[0, 1] * t3_ref[...]


def _combine(t3, Sp, g, degc, b, scal):
    grid = (_N // _BM,)
    return pl.pallas_call(
        _combine_kernel,
        grid=grid,
        in_specs=[
            pl.BlockSpec((_BM, _D), lambda i: (i, 0)),
            pl.BlockSpec((_BM, _D), lambda i: (i, 0)),
            pl.BlockSpec((_BM, _D), lambda i: (i, 0)),
            pl.BlockSpec((_BM, _DEGW), lambda i: (i, 0)),
            pl.BlockSpec((1, _D), lambda i: (0, 0)),
            pl.BlockSpec((1, 2), lambda i: (0, 0)),
        ],
        out_specs=pl.BlockSpec((_BM, _D), lambda i: (i, 0)),
        out_shape=jax.ShapeDtypeStruct((_N, _D), jnp.float32),
    )(t3, Sp, g, degc, b, scal)


def kernel(x, edge_index, lap, d_inv, W_high, W_conv, b_conv, aL, aH):
    src2 = edge_index[0]
    dst2 = edge_index[1]
    dst3 = dst2.reshape(_NQ, _QB, _CH)
    degp = _deg_counts(dst3)
    g, H = _prep(x, W_conv, W_high, degp)
    Sp = _msg_sums(src2, dst2, g)
    t1 = _mm(d_inv, H)
    t2 = _mm(lap, t1)
    t3 = _mm(d_inv, t2)
    scal = jnp.concatenate([aL, aH]).reshape(1, 2)
    return _combine(t3, Sp, g, degp, b_conv.reshape(1, _D), scal)
